# baseline (speedup 1.0000x reference)
import functools
import jax
import jax.numpy as jnp
from jax.experimental import pallas as pl


def _mm_kernel(x_ref, w_ref, b_ref, o_ref):
    o_ref[...] = (
        jnp.dot(x_ref[...], w_ref[...], preferred_element_type=jnp.float32)
        + b_ref[...]
    )


def _fused_proj(x, wall_t, ball, bn=400):
    n, din = x.shape
    c = wall_t.shape[1]
    assert n % bn == 0
    return pl.pallas_call(
        _mm_kernel,
        grid=(n // bn,),
        in_specs=[
            pl.BlockSpec((bn, din), lambda i: (i, 0)),
            pl.BlockSpec((din, c), lambda i: (0, 0)),
            pl.BlockSpec((1, c), lambda i: (0, 0)),
        ],
        out_specs=pl.BlockSpec((bn, c), lambda i: (i, 0)),
        out_shape=jax.ShapeDtypeStruct((n, c), jnp.float32),
    )(x, wall_t, ball)


def _conv(x, src, dst, ea, Wq, bq, Wk, bk, Wv, bv, We, Ws, bs):
    n = x.shape[0]
    d = Wq.shape[0]
    wall_t = jnp.concatenate([Wq, Wk, Wv, Ws], axis=0).T
    ball = jnp.concatenate([bq, bk, bv, bs])[None, :]
    proj = _fused_proj(x, wall_t, ball)
    q = proj[:, 0 * d:1 * d]
    k = proj[:, 1 * d:2 * d]
    v = proj[:, 2 * d:3 * d]
    skip = proj[:, 3 * d:4 * d]

    we = We[:, 0]
    qwe = q @ we
    scale = 1.0 / jnp.sqrt(jnp.float32(d))
    alpha = (jnp.sum(q[dst] * k[src], axis=-1) + ea * qwe[dst]) * scale
    amax = jax.ops.segment_max(alpha, dst, num_segments=n)
    amax = jnp.where(jnp.isfinite(amax), amax, 0.0)
    ex = jnp.exp(alpha - amax[dst])
    den = jax.ops.segment_sum(ex, dst, num_segments=n)
    num1 = jax.ops.segment_sum(ex[:, None] * v[src], dst, num_segments=n)
    wsum = jax.ops.segment_sum(ex * ea, dst, num_segments=n)
    inv = 1.0 / (den + 1e-16)
    agg = (num1 + wsum[:, None] * we[None, :]) * inv[:, None]
    return agg + skip


def kernel(x, edge_index, edge_attr, Wq1, bq1, Wk1, bk1, Wv1, bv1, We1, Ws1, bs1, Wq2, bq2, Wk2, bk2, Wv2, bv2, We2, Ws2, bs2):
    src = edge_index[0]
    dst = edge_index[1]
    ea = edge_attr[:, 0]
    h = jax.nn.relu(_conv(x, src, dst, ea, Wq1, bq1, Wk1, bk1, Wv1, bv1, We1, Ws1, bs1))
    return _conv(h, src, dst, ea, Wq2, bq2, Wk2, bk2, Wv2, bv2, We2, Ws2, bs2)


# speedup vs baseline: 1.2222x; 1.2222x over previous
import functools
import jax
import jax.numpy as jnp
from jax import lax
from jax.experimental import pallas as pl
from jax.experimental.pallas import tpu as pltpu, tpu_sc as plsc

_NC = 2
_NS = 16
_NW = _NC * _NS


def _sc_gather3(q, k, v, dstv, srcv):
    e = dstv.shape[0]
    d = q.shape[1]
    per_w = e // _NW
    cb = 40
    mesh = plsc.VectorSubcoreMesh(
        core_axis_name="c", subcore_axis_name="s",
        num_cores=_NC, num_subcores=_NS)

    @functools.partial(
        pl.kernel, mesh=mesh,
        out_type=[jax.ShapeDtypeStruct((e, d), jnp.float32)] * 3,
        scratch_types=[
            pltpu.VMEM((cb,), jnp.int32),
            pltpu.VMEM((cb,), jnp.int32),
            pltpu.VMEM((cb, d), jnp.float32),
            pltpu.VMEM((cb, d), jnp.float32),
            pltpu.VMEM((cb, d), jnp.float32),
            pltpu.SemaphoreType.DMA,
        ],
    )
    def kk(q_hbm, k_hbm, v_hbm, dst_hbm, src_hbm, oq, ok, ov,
           di, si, qb, kb, vb, sem):
        wid = lax.axis_index("s") * _NC + lax.axis_index("c")
        base = wid * per_w

        def body(i, carry):
            off = base + i * cb
            pltpu.sync_copy(dst_hbm.at[pl.ds(off, cb)], di)
            pltpu.sync_copy(src_hbm.at[pl.ds(off, cb)], si)
            pltpu.async_copy(q_hbm.at[di], qb, sem).wait()
            pltpu.async_copy(k_hbm.at[si], kb, sem).wait()
            pltpu.async_copy(v_hbm.at[si], vb, sem).wait()
            pltpu.sync_copy(qb, oq.at[pl.ds(off, cb)])
            pltpu.sync_copy(kb, ok.at[pl.ds(off, cb)])
            pltpu.sync_copy(vb, ov.at[pl.ds(off, cb)])
            return carry

        lax.fori_loop(0, per_w // cb, body, 0)

    return kk(q, k, v, dstv, srcv)


def _dot_kernel(q_ref, k_ref, o_ref):
    o_ref[...] = jnp.sum(q_ref[...] * k_ref[...], axis=1)


def _edge_dot(qd, ks, be=256):
    e, d = qd.shape
    return pl.pallas_call(
        _dot_kernel,
        grid=(e // be,),
        in_specs=[
            pl.BlockSpec((be, d), lambda i: (i, 0)),
            pl.BlockSpec((be, d), lambda i: (i, 0)),
        ],
        out_specs=pl.BlockSpec((be,), lambda i: (i,)),
        out_shape=jax.ShapeDtypeStruct((e,), jnp.float32),
    )(qd, ks)


def _mm_kernel(x_ref, w_ref, b_ref, o_ref):
    o_ref[...] = (
        jnp.dot(x_ref[...], w_ref[...], preferred_element_type=jnp.float32)
        + b_ref[...]
    )


def _fused_proj(x, wall_t, ball, bn=400):
    n, din = x.shape
    c = wall_t.shape[1]
    assert n % bn == 0
    return pl.pallas_call(
        _mm_kernel,
        grid=(n // bn,),
        in_specs=[
            pl.BlockSpec((bn, din), lambda i: (i, 0)),
            pl.BlockSpec((din, c), lambda i: (0, 0)),
            pl.BlockSpec((1, c), lambda i: (0, 0)),
        ],
        out_specs=pl.BlockSpec((bn, c), lambda i: (i, 0)),
        out_shape=jax.ShapeDtypeStruct((n, c), jnp.float32),
    )(x, wall_t, ball)


def _conv(x, src, dst, ea, Wq, bq, Wk, bk, Wv, bv, We, Ws, bs):
    n = x.shape[0]
    d = Wq.shape[0]
    wall_t = jnp.concatenate([Wq, Wk, Wv, Ws], axis=0).T
    ball = jnp.concatenate([bq, bk, bv, bs])[None, :]
    proj = _fused_proj(x, wall_t, ball)
    q = proj[:, 0 * d:1 * d]
    k = proj[:, 1 * d:2 * d]
    v = proj[:, 2 * d:3 * d]
    skip = proj[:, 3 * d:4 * d]

    we = We[:, 0]
    qwe = q @ we
    scale = 1.0 / jnp.sqrt(jnp.float32(d))
    qd, ks, vs = _sc_gather3(q, k, v, dst, src)
    alpha = (_edge_dot(qd, ks) + ea * qwe[dst]) * scale
    amax = jax.ops.segment_max(alpha, dst, num_segments=n)
    amax = jnp.where(jnp.isfinite(amax), amax, 0.0)
    ex = jnp.exp(alpha - amax[dst])
    den = jax.ops.segment_sum(ex, dst, num_segments=n)
    num1 = jax.ops.segment_sum(ex[:, None] * vs, dst, num_segments=n)
    wsum = jax.ops.segment_sum(ex * ea, dst, num_segments=n)
    inv = 1.0 / (den + 1e-16)
    agg = (num1 + wsum[:, None] * we[None, :]) * inv[:, None]
    return agg + skip


def kernel(x, edge_index, edge_attr, Wq1, bq1, Wk1, bk1, Wv1, bv1, We1, Ws1, bs1, Wq2, bq2, Wk2, bk2, Wv2, bv2, We2, Ws2, bs2):
    src = edge_index[0]
    dst = edge_index[1]
    ea = edge_attr[:, 0]
    h = jax.nn.relu(_conv(x, src, dst, ea, Wq1, bq1, Wk1, bk1, Wv1, bv1, We1, Ws1, bs1))
    return _conv(h, src, dst, ea, Wq2, bq2, Wk2, bk2, Wv2, bv2, We2, Ws2, bs2)
